# Initial kernel scaffold; baseline (speedup 1.0000x reference)
import jax
import jax.numpy as jnp
from jax.experimental import pallas as pl


def kernel(table_feat, row_graph, col_graph, W_r1, b_r1, W_c1, b_c1, Ws_r, bs_r, g_r, be_r, Ws_c, bs_c, g_c, be_c, Wm, bm, gm, bem, W_r2, b_r2, W_c2, b_c2, w1, b1, w2, b2):
    raise NotImplementedError("write your pallas kernel here")


# speedup vs baseline: 3.4841x; 3.4841x over previous
import dataclasses
import functools

import jax
import jax.numpy as jnp
from jax import lax
from jax.experimental import pallas as pl
from jax.experimental.pallas import tpu as pltpu
from jax.experimental.pallas import tpu_sc as plsc

N = 10000
E = 160000
D = 300
DH = 150
DHP = 160
NPAD = 10240
NW = 32
EPW = E // NW
CHUNK = 80
ROWS_PER_SUB = E // 16 // CHUNK
NBLK = 5
RPB = ROWS_PER_SUB // NBLK
NSUB = 16
NDRAIN = NPAD // NSUB

_vmesh = plsc.VectorSubcoreMesh(core_axis_name="c", subcore_axis_name="s")

_sc_params = pltpu.CompilerParams()
if "needs_layout_passes" in pltpu.CompilerParams.__dataclass_fields__:
    _sc_params = dataclasses.replace(_sc_params, needs_layout_passes=False)
_sc_agg_params = dataclasses.replace(_sc_params, use_tc_tiling_on_sc=False)


def _sc_degrees(idx_flat):
    @functools.partial(
        pl.kernel,
        out_type=jax.ShapeDtypeStruct((NW, 4, NPAD), jnp.float32),
        mesh=_vmesh,
        compiler_params=_sc_params,
        scratch_types=[
            pltpu.VMEM((EPW,), jnp.int32),
            pltpu.VMEM((NPAD,), jnp.float32),
        ],
    )
    def k(idx_hbm, out_hbm, idx_v, hist_v):
        wid = lax.axis_index("s") * 2 + lax.axis_index("c")
        ones = jnp.full((16,), 1.0, jnp.float32)
        zeros = jnp.zeros((16,), jnp.float32)
        for h in range(4):
            pltpu.sync_copy(idx_hbm.at[h].at[wid], idx_v)

            @pl.loop(0, NPAD // 16)
            def _(t):
                hist_v[pl.ds(t * 16, 16)] = zeros

            @pl.loop(0, EPW // 16)
            def _(t):
                idx16 = idx_v[pl.ds(t * 16, 16)]
                plsc.addupdate_scatter(hist_v, [idx16], ones)

            pltpu.sync_copy(hist_v, out_hbm.at[wid, h])

    return k(idx_flat)


def _tc_scales(deg_part):
    def body(d_ref, o_ref):
        deg = jnp.sum(d_ref[...], axis=0)
        s = lax.rsqrt(jnp.clip(deg, 1.0, None))
        o_ref[...] = s.T

    return pl.pallas_call(
        body,
        out_shape=jax.ShapeDtypeStruct((NPAD, 4), jnp.float32),
    )(deg_part)


def _split_pad(h):
    h0 = jnp.pad(h[:, :DH], ((0, 0), (0, DHP - DH)))
    h1 = jnp.pad(h[:, DH:], ((0, 0), (0, DHP - DH)))
    return jnp.stack([h0, h1])


BN = 400


def _tc_pre(x, W_r1, W_c1, scales):
    def body(x_ref, wr_ref, wc_ref, s_ref, hr_ref, hc_ref):
        xb = x_ref[...]
        s_or = s_ref[:, 0:1]
        s_oc = s_ref[:, 2:3]
        hr = jnp.dot(xb, wr_ref[...], preferred_element_type=jnp.float32) * s_or
        hc = jnp.dot(xb, wc_ref[...], preferred_element_type=jnp.float32) * s_oc
        hr_ref[...] = _split_pad(hr)
        hc_ref[...] = _split_pad(hc)

    grid = (N // BN,)
    return pl.pallas_call(
        body,
        grid=grid,
        in_specs=[
            pl.BlockSpec((BN, D), lambda i: (i, 0)),
            pl.BlockSpec((D, D), lambda i: (0, 0)),
            pl.BlockSpec((D, D), lambda i: (0, 0)),
            pl.BlockSpec((BN, 4), lambda i: (i, 0)),
        ],
        out_specs=[
            pl.BlockSpec((2, BN, DHP), lambda i: (0, i, 0)),
            pl.BlockSpec((2, BN, DHP), lambda i: (0, i, 0)),
        ],
        out_shape=[
            jax.ShapeDtypeStruct((2, N, DHP), jnp.float32),
            jax.ShapeDtypeStruct((2, N, DHP), jnp.float32),
        ],
    )(x, W_r1, W_c1, scales)


def _sc_aggregate(h_r, h_c, rs2, rd2, cs2, cd2, zrows):
    out_t = jax.ShapeDtypeStruct((2, NPAD, DHP), jnp.float32)

    @functools.partial(
        pl.kernel,
        out_type=(out_t, out_t),
        mesh=_vmesh,
        compiler_params=_sc_agg_params,
        scratch_types=[
            pltpu.VMEM_SHARED((NPAD, DHP), jnp.float32),
            pltpu.VMEM((RPB, CHUNK), jnp.int32),
            pltpu.VMEM((RPB, CHUNK), jnp.int32),
            pltpu.VMEM((CHUNK,), jnp.int32),
            pltpu.VMEM((CHUNK, DHP), jnp.float32),
            pltpu.SemaphoreType.DMA,
        ],
    )
    def k(hr_hbm, hc_hbm, rs_hbm, rd_hbm, cs_hbm, cd_hbm, z_hbm,
          ar_hbm, ac_hbm, acc, src_v, dst_v, dbuf, gbuf, sem):
        c = lax.axis_index("c")
        s = lax.axis_index("s")
        for (h_hbm, s_hbm, d_hbm, o_hbm) in (
            (hr_hbm, rs_hbm, rd_hbm, ar_hbm),
            (hc_hbm, cs_hbm, cd_hbm, ac_hbm),
        ):
            pltpu.sync_copy(
                z_hbm, acc.at[pl.ds(pl.multiple_of(s * NDRAIN, NDRAIN),
                                    NDRAIN)])
            plsc.subcore_barrier()

            @pl.loop(0, NBLK)
            def _(b):
                pltpu.sync_copy(s_hbm.at[s].at[b], src_v)
                pltpu.sync_copy(d_hbm.at[s].at[b], dst_v)

                @pl.loop(0, RPB)
                def _(j):
                    for t in range(CHUNK // 16):
                        dbuf[pl.ds(t * 16, 16)] = dst_v[j, pl.ds(t * 16, 16)]
                    pltpu.async_copy(h_hbm.at[c].at[src_v.at[j]],
                                     gbuf, sem).wait()
                    pltpu.sync_copy(gbuf, acc.at[dbuf], add=True)

            plsc.subcore_barrier()
            drain = pl.ds(pl.multiple_of(s * NDRAIN, NDRAIN), NDRAIN)
            pltpu.sync_copy(acc.at[drain], o_hbm.at[c].at[drain])
            plsc.subcore_barrier()

    return k(h_r, h_c, rs2, rd2, cs2, cd2, zrows)


def _merge_halves(a_ref):
    a = a_ref[...]
    return jnp.concatenate([a[0, :, :DH], a[1, :, :DH]], axis=1)


def _ln(x, g, b):
    m = jnp.mean(x, axis=-1, keepdims=True)
    v = jnp.mean((x - m) ** 2, axis=-1, keepdims=True)
    return (x - m) * lax.rsqrt(v + 1e-5) * g + b


def _tc_mid(agg_r, agg_c, scales, b_r1, b_c1,
            Ws_r, bs_r, g_r, be_r, Ws_c, bs_c, g_c, be_c,
            Wm, bm, gm, bem, W_r2, W_c2):
    def body(ar_ref, ac_ref, s_ref, br1_ref, bc1_ref,
             wsr_ref, bsr_ref, gr_ref, ber_ref,
             wsc_ref, bsc_ref, gc_ref, bec_ref,
             wm_ref, bm_ref, gm_ref, bem_ref,
             wr2_ref, wc2_ref, hr_ref, hc_ref):
        s_or = s_ref[:, 0:1]
        s_ir = s_ref[:, 1:2]
        s_oc = s_ref[:, 2:3]
        s_ic = s_ref[:, 3:4]
        r1 = jax.nn.relu(_merge_halves(ar_ref) * s_ir + br1_ref[...])
        c1 = jax.nn.relu(_merge_halves(ac_ref) * s_ic + bc1_ref[...])
        rs = _ln(jnp.dot(r1, wsr_ref[...], preferred_element_type=jnp.float32)
                 + bsr_ref[...], gr_ref[...], ber_ref[...])
        cs = _ln(jnp.dot(c1, wsc_ref[...], preferred_element_type=jnp.float32)
                 + bsc_ref[...], gc_ref[...], bec_ref[...])
        cat = jnp.concatenate([rs, cs], axis=1)
        g_rep = _ln(jnp.dot(cat, wm_ref[...],
                            preferred_element_type=jnp.float32)
                    + bm_ref[...], gm_ref[...], bem_ref[...])
        hr = jnp.dot(g_rep, wr2_ref[...],
                     preferred_element_type=jnp.float32) * s_or
        hc = jnp.dot(g_rep, wc2_ref[...],
                     preferred_element_type=jnp.float32) * s_oc
        hr_ref[...] = _split_pad(hr)
        hc_ref[...] = _split_pad(hc)

    vec = lambda: pl.BlockSpec((D,), lambda i: (0,))
    mat = lambda r: pl.BlockSpec((r, D), lambda i: (0, 0))
    grid = (N // BN,)
    return pl.pallas_call(
        body,
        grid=grid,
        in_specs=[
            pl.BlockSpec((2, BN, DHP), lambda i: (0, i, 0)),
            pl.BlockSpec((2, BN, DHP), lambda i: (0, i, 0)),
            pl.BlockSpec((BN, 4), lambda i: (i, 0)),
            vec(), vec(),
            mat(D), vec(), vec(), vec(),
            mat(D), vec(), vec(), vec(),
            mat(2 * D), vec(), vec(), vec(),
            mat(D), mat(D),
        ],
        out_specs=[
            pl.BlockSpec((2, BN, DHP), lambda i: (0, i, 0)),
            pl.BlockSpec((2, BN, DHP), lambda i: (0, i, 0)),
        ],
        out_shape=[
            jax.ShapeDtypeStruct((2, N, DHP), jnp.float32),
            jax.ShapeDtypeStruct((2, N, DHP), jnp.float32),
        ],
    )(agg_r, agg_c, scales, b_r1, b_c1,
      Ws_r, bs_r, g_r, be_r, Ws_c, bs_c, g_c, be_c,
      Wm, bm, gm, bem, W_r2, W_c2)


def _tc_final(agg_r, agg_c, scales, b_r2, b_c2, w1, b1, w2, b2):
    def body(ar_ref, ac_ref, s_ref, br2_ref, bc2_ref,
             w1_ref, b1_ref, w2_ref, b2_ref, o_ref):
        s_ir = s_ref[:, 1:2]
        s_ic = s_ref[:, 3:4]
        r2 = jax.nn.relu(_merge_halves(ar_ref) * s_ir + br2_ref[...])
        c2 = jax.nn.relu(_merge_halves(ac_ref) * s_ic + bc2_ref[...])
        alpha = jax.nn.sigmoid(
            jnp.dot(r2, w1_ref[...], preferred_element_type=jnp.float32)
            + b1_ref[...])
        beta = jax.nn.sigmoid(
            jnp.dot(c2, w2_ref[...], preferred_element_type=jnp.float32)
            + b2_ref[...])
        a = alpha / (alpha + beta)
        o_ref[...] = a * r2 + (1.0 - a) * c2

    grid = (N // BN,)
    return pl.pallas_call(
        body,
        grid=grid,
        in_specs=[
            pl.BlockSpec((2, BN, DHP), lambda i: (0, i, 0)),
            pl.BlockSpec((2, BN, DHP), lambda i: (0, i, 0)),
            pl.BlockSpec((BN, 4), lambda i: (i, 0)),
            pl.BlockSpec((D,), lambda i: (0,)),
            pl.BlockSpec((D,), lambda i: (0,)),
            pl.BlockSpec((D, 1), lambda i: (0, 0)),
            pl.BlockSpec((1,), lambda i: (0,)),
            pl.BlockSpec((D, 1), lambda i: (0, 0)),
            pl.BlockSpec((1,), lambda i: (0,)),
        ],
        out_specs=pl.BlockSpec((BN, D), lambda i: (i, 0)),
        out_shape=jax.ShapeDtypeStruct((N, D), jnp.float32),
    )(agg_r, agg_c, scales, b_r2, b_c2, w1, b1, w2, b2)


def kernel(table_feat, row_graph, col_graph, W_r1, b_r1, W_c1, b_c1,
           Ws_r, bs_r, g_r, be_r, Ws_c, bs_c, g_c, be_c,
           Wm, bm, gm, bem, W_r2, b_r2, W_c2, b_c2, w1, b1, w2, b2):
    idx_flat = jnp.concatenate([row_graph, col_graph],
                               axis=0).reshape(4, NW, EPW)
    rs2 = row_graph[0].reshape(NSUB, NBLK, RPB, CHUNK)
    rd2 = row_graph[1].reshape(NSUB, NBLK, RPB, CHUNK)
    cs2 = col_graph[0].reshape(NSUB, NBLK, RPB, CHUNK)
    cd2 = col_graph[1].reshape(NSUB, NBLK, RPB, CHUNK)
    zrows = jnp.zeros((NDRAIN, DHP), jnp.float32)

    deg_part = _sc_degrees(idx_flat)
    scales = _tc_scales(deg_part)
    h_r, h_c = _tc_pre(table_feat, W_r1, W_c1, scales)
    agg_r, agg_c = _sc_aggregate(h_r, h_c, rs2, rd2, cs2, cd2, zrows)
    h2_r, h2_c = _tc_mid(agg_r, agg_c, scales, b_r1, b_c1,
                         Ws_r, bs_r, g_r, be_r, Ws_c, bs_c, g_c, be_c,
                         Wm, bm, gm, bem, W_r2, W_c2)
    agg2_r, agg2_c = _sc_aggregate(h2_r, h2_c, rs2, rd2, cs2, cd2, zrows)
    return _tc_final(agg2_r, agg2_c, scales, b_r2, b_c2, w1, b1, w2, b2)
